# baseline (speedup 1.0000x reference)
import functools

import jax
import jax.numpy as jnp
from jax import lax
from jax.experimental import pallas as pl
from jax.experimental.pallas import tpu as pltpu
from jax.experimental.pallas import tpu_sc as plsc

N_NODES = 10000
N_EDGES = 320000
D = 128

NC = 2
NS = 16
L = 16
NW = NC * NS
FPW = D // NW
CHUNK = 6400
N_CHUNKS = N_EDGES // CHUNK


def _sc_agg_body(xt_hbm, edges_hbm, aggt_hbm,
                 xtv, xv, accv, eb0, eb1, sem0, sem1):
    c = lax.axis_index("c")
    s = lax.axis_index("s")
    wid = c * NS + s
    fbase = wid * FPW

    def fetch(ci, eb, sem):
        pltpu.make_async_copy(
            edges_hbm.at[pl.ds(ci * CHUNK, CHUNK)], eb, sem).start()

    def drain(eb, sem):
        pltpu.make_async_copy(
            edges_hbm.at[pl.ds(0, CHUNK)], eb, sem).wait()

    fetch(0, eb0, sem0)
    fetch(1, eb1, sem1)
    pltpu.sync_copy(xt_hbm.at[pl.ds(fbase, FPW)], xtv)

    zero16 = jnp.zeros((L,), jnp.float32)

    @plsc.parallel_loop(0, N_NODES // L, unroll=8)
    def _zero(i):
        for j in range(FPW):
            accv[j, pl.ds(i * L, L)] = zero16

    @plsc.parallel_loop(0, N_NODES // L, unroll=8)
    def _pack(i):
        for p in range(FPW // 2):
            a = xtv[2 * p, pl.ds(i * L, L)]
            bvals = xtv[2 * p + 1, pl.ds(i * L, L)]
            packed = plsc.pack(a, bvals, format=plsc.PackFormat.INTERLEAVED)
            xv[p, pl.ds(i * L, L)] = plsc.bitcast(packed, jnp.int32)

    def process(eb):
        @plsc.parallel_loop(0, CHUNK // L, unroll=16)
        def _steps(i):
            w16 = eb[pl.ds(i * L, L)]
            col16, row16 = plsc.unpack(plsc.bitcast(w16, jnp.int16),
                                       format=plsc.PackFormat.INTERLEAVED,
                                       preferred_element_type=jnp.int32)
            for p in range(FPW // 2):
                pv = jnp.full((L,), p, jnp.int32)
                w = plsc.load_gather(xv, [pv, col16])
                lo, hi = plsc.unpack(plsc.bitcast(w, jnp.bfloat16),
                                     format=plsc.PackFormat.INTERLEAVED)
                j0 = jnp.full((L,), 2 * p, jnp.int32)
                j1 = jnp.full((L,), 2 * p + 1, jnp.int32)
                plsc.addupdate_scatter(accv, [j0, row16], lo)
                plsc.addupdate_scatter(accv, [j1, row16], hi)

    bufs = ((eb0, sem0), (eb1, sem1))

    def pair_body(k, _):
        for b in range(2):
            ci = k * 2 + b
            eb, sem = bufs[b]
            drain(eb, sem)
            process(eb)

            @pl.when(ci + 2 < N_CHUNKS)
            def _():
                fetch(ci + 2, eb, sem)
        return 0

    lax.fori_loop(0, N_CHUNKS // 2, pair_body, 0)

    pltpu.sync_copy(accv, aggt_hbm.at[pl.ds(fbase, FPW)])


_sc_agg = functools.partial(
    pl.kernel,
    out_type=jax.ShapeDtypeStruct((D, N_NODES), jnp.float32),
    mesh=plsc.VectorSubcoreMesh(core_axis_name="c", subcore_axis_name="s"),
    compiler_params=pltpu.CompilerParams(use_tc_tiling_on_sc=False,
                                         needs_layout_passes=False),
    scratch_types=[
        pltpu.VMEM((FPW, N_NODES), jnp.float32),
        pltpu.VMEM((FPW // 2, N_NODES), jnp.int32),
        pltpu.VMEM((FPW, N_NODES), jnp.float32),
        pltpu.VMEM((CHUNK,), jnp.int32),
        pltpu.VMEM((CHUNK,), jnp.int32),
        pltpu.SemaphoreType.DMA,
        pltpu.SemaphoreType.DMA,
    ],
)(_sc_agg_body)


def _tc_edgepack_body(e_ref, o_ref):
    e = e_ref[...]
    o_ref[...] = (e[0] << 16) | e[1]


def _tc_edgepack(edge_index):
    return pl.pallas_call(
        _tc_edgepack_body,
        out_shape=jax.ShapeDtypeStruct((N_EDGES,), jnp.int32),
    )(edge_index)


def _tc_linear_body(aggt_ref, w_ref, b_ref, o_ref):
    a = aggt_ref[...]
    w = w_ref[...]
    acc = lax.dot_general(a, w, (((0,), (1,)), ((), ())),
                          preferred_element_type=jnp.float32)
    o_ref[...] = jnp.maximum(acc + b_ref[...], 0.0)


def _tc_linear(aggt, w, b2d):
    return pl.pallas_call(
        _tc_linear_body,
        out_shape=jax.ShapeDtypeStruct((N_NODES, D), jnp.float32),
    )(aggt, w, b2d)


def kernel(x, edge_index, W, b):
    xt = x.T
    epk = _tc_edgepack(edge_index)
    aggt = _sc_agg(xt, epk)
    return _tc_linear(aggt, W, b.reshape(1, D))


# speedup vs baseline: 1.0302x; 1.0302x over previous
import functools

import jax
import jax.numpy as jnp
from jax import lax
from jax.experimental import pallas as pl
from jax.experimental.pallas import tpu as pltpu
from jax.experimental.pallas import tpu_sc as plsc

N_NODES = 10000
N_EDGES = 320000
D = 128

NC = 2
NS = 16
L = 16
NW = NC * NS
FPW = D // NW
CHUNK = 6400
N_CHUNKS = N_EDGES // CHUNK


def _sc_agg_body(xt_hbm, edges_hbm, aggt_hbm,
                 xtv, xv, accv, eb0, eb1, sem0, sem1):
    c = lax.axis_index("c")
    s = lax.axis_index("s")
    wid = c * NS + s
    fbase = wid * FPW

    def fetch(ci, eb, sem):
        pltpu.make_async_copy(
            edges_hbm.at[pl.ds(ci * CHUNK, CHUNK)], eb, sem).start()

    def drain(eb, sem):
        pltpu.make_async_copy(
            edges_hbm.at[pl.ds(0, CHUNK)], eb, sem).wait()

    fetch(0, eb0, sem0)
    fetch(1, eb1, sem1)
    pltpu.sync_copy(xt_hbm.at[pl.ds(fbase, FPW)], xtv)

    zero16 = jnp.zeros((L,), jnp.float32)

    @plsc.parallel_loop(0, N_NODES // L, unroll=8)
    def _zero(i):
        for j in range(FPW):
            accv[j, pl.ds(i * L, L)] = zero16

    @plsc.parallel_loop(0, N_NODES // L, unroll=8)
    def _pack(i):
        for p in range(FPW // 2):
            a = xtv[2 * p, pl.ds(i * L, L)]
            bvals = xtv[2 * p + 1, pl.ds(i * L, L)]
            packed = plsc.pack(a, bvals, format=plsc.PackFormat.INTERLEAVED)
            xv[p, pl.ds(i * L, L)] = plsc.bitcast(packed, jnp.int32)

    def process(eb):
        @plsc.parallel_loop(0, CHUNK // L, unroll=8)
        def _steps(i):
            w16 = eb[pl.ds(i * L, L)]
            col16, row16 = plsc.unpack(plsc.bitcast(w16, jnp.int16),
                                       format=plsc.PackFormat.INTERLEAVED,
                                       preferred_element_type=jnp.int32)
            for p in range(FPW // 2):
                pv = jnp.full((L,), p, jnp.int32)
                w = plsc.load_gather(xv, [pv, col16])
                lo, hi = plsc.unpack(plsc.bitcast(w, jnp.bfloat16),
                                     format=plsc.PackFormat.INTERLEAVED)
                j0 = jnp.full((L,), 2 * p, jnp.int32)
                j1 = jnp.full((L,), 2 * p + 1, jnp.int32)
                plsc.addupdate_scatter(accv, [j0, row16], lo)
                plsc.addupdate_scatter(accv, [j1, row16], hi)

    bufs = ((eb0, sem0), (eb1, sem1))

    def pair_body(k, _):
        for b in range(2):
            ci = k * 2 + b
            eb, sem = bufs[b]
            drain(eb, sem)
            process(eb)

            @pl.when(ci + 2 < N_CHUNKS)
            def _():
                fetch(ci + 2, eb, sem)
        return 0

    lax.fori_loop(0, N_CHUNKS // 2, pair_body, 0)

    pltpu.sync_copy(accv, aggt_hbm.at[pl.ds(fbase, FPW)])


_sc_agg = functools.partial(
    pl.kernel,
    out_type=jax.ShapeDtypeStruct((D, N_NODES), jnp.float32),
    mesh=plsc.VectorSubcoreMesh(core_axis_name="c", subcore_axis_name="s"),
    compiler_params=pltpu.CompilerParams(use_tc_tiling_on_sc=False,
                                         needs_layout_passes=False),
    scratch_types=[
        pltpu.VMEM((FPW, N_NODES), jnp.float32),
        pltpu.VMEM((FPW // 2, N_NODES), jnp.int32),
        pltpu.VMEM((FPW, N_NODES), jnp.float32),
        pltpu.VMEM((CHUNK,), jnp.int32),
        pltpu.VMEM((CHUNK,), jnp.int32),
        pltpu.SemaphoreType.DMA,
        pltpu.SemaphoreType.DMA,
    ],
)(_sc_agg_body)


def _tc_edgepack_body(e_ref, o_ref):
    e = e_ref[...]
    o_ref[...] = (e[0] << 16) | e[1]


def _tc_edgepack(edge_index):
    return pl.pallas_call(
        _tc_edgepack_body,
        out_shape=jax.ShapeDtypeStruct((N_EDGES,), jnp.int32),
    )(edge_index)


def _tc_linear_body(aggt_ref, w_ref, b_ref, o_ref):
    a = aggt_ref[...]
    w = w_ref[...]
    acc = lax.dot_general(a, w, (((0,), (1,)), ((), ())),
                          preferred_element_type=jnp.float32)
    o_ref[...] = jnp.maximum(acc + b_ref[...], 0.0)


def _tc_linear(aggt, w, b2d):
    return pl.pallas_call(
        _tc_linear_body,
        out_shape=jax.ShapeDtypeStruct((N_NODES, D), jnp.float32),
    )(aggt, w, b2d)


def kernel(x, edge_index, W, b):
    xt = x.T
    epk = _tc_edgepack(edge_index)
    aggt = _sc_agg(xt, epk)
    return _tc_linear(aggt, W, b.reshape(1, D))
